# Initial kernel scaffold; baseline (speedup 1.0000x reference)
import jax
import jax.numpy as jnp
from jax.experimental import pallas as pl


def kernel(features, edge_index, W1, b1, W2, b2):
    raise NotImplementedError("write your pallas kernel here")


# speedup vs baseline: 3.4392x; 3.4392x over previous
import functools

import jax
import jax.numpy as jnp
from jax import lax
from jax.experimental import pallas as pl
from jax.experimental.pallas import tpu as pltpu
from jax.experimental.pallas import tpu_sc as plsc

N = 10000
NP = 10240
E = 320000
EP = 327680
CHUNK = 128
CPT = EP // (32 * CHUNK)
ROWS_PT = NP // 16
PAD_IDX = NP - 1
DW = 16

_mesh = plsc.VectorSubcoreMesh(core_axis_name="c", subcore_axis_name="s")


def _deg_body(src_hbm, dst_hbm, ones_hbm, out_hbm,
              src_v, dst_v, ones_v, acc_s, acc_d):
    cid = lax.axis_index("c")
    sid = lax.axis_index("s")
    wid = sid * 2 + cid
    pltpu.sync_copy(src_hbm.at[pl.ds(wid * CPT, CPT)], src_v)
    pltpu.sync_copy(dst_hbm.at[pl.ds(wid * CPT, CPT)], dst_v)
    pltpu.sync_copy(ones_hbm, ones_v)
    for j in range(ROWS_PT // CHUNK):
        pltpu.sync_copy(ones_hbm.at[pl.ds(CHUNK, CHUNK)],
                        acc_s.at[pl.ds(sid * ROWS_PT + j * CHUNK, CHUNK)])
        pltpu.sync_copy(ones_hbm.at[pl.ds(CHUNK, CHUNK)],
                        acc_d.at[pl.ds(sid * ROWS_PT + j * CHUNK, CHUNK)])
    plsc.subcore_barrier()

    def chunk(c, carry):
        pltpu.sync_copy(ones_v.at[pl.ds(0, CHUNK)], acc_s.at[src_v.at[c]],
                        add=True)
        pltpu.sync_copy(ones_v.at[pl.ds(0, CHUNK)], acc_d.at[dst_v.at[c]],
                        add=True)
        return carry

    lax.fori_loop(0, CPT, chunk, 0)
    plsc.subcore_barrier()
    r0 = sid * ROWS_PT
    pltpu.sync_copy(acc_s.at[pl.ds(r0, ROWS_PT)],
                    out_hbm.at[cid, 0].at[pl.ds(r0, ROWS_PT)])
    pltpu.sync_copy(acc_d.at[pl.ds(r0, ROWS_PT)],
                    out_hbm.at[cid, 1].at[pl.ds(r0, ROWS_PT)])


_sc_params = pltpu.CompilerParams(use_tc_tiling_on_sc=False)

_deg_kernel = functools.partial(
    pl.kernel,
    out_type=jax.ShapeDtypeStruct((2, 2, NP, DW), jnp.float32),
    mesh=_mesh,
    compiler_params=_sc_params,
    scratch_types=[
        pltpu.VMEM((CPT, CHUNK), jnp.int32),
        pltpu.VMEM((CPT, CHUNK), jnp.int32),
        pltpu.VMEM((2 * CHUNK, DW), jnp.float32),
        pltpu.VMEM_SHARED((NP, DW), jnp.float32),
        pltpu.VMEM_SHARED((NP, DW), jnp.float32),
    ],
)(_deg_body)


def _spmm_body(x_hbm, src_hbm, dst_hbm, zeros_hbm, out_hbm,
               src_v, dst_v, rows_v, acc):
    cid = lax.axis_index("c")
    sid = lax.axis_index("s")
    wid = sid * 2 + cid
    pltpu.sync_copy(src_hbm.at[pl.ds(wid * CPT, CPT)], src_v)
    pltpu.sync_copy(dst_hbm.at[pl.ds(wid * CPT, CPT)], dst_v)
    pltpu.sync_copy(zeros_hbm, rows_v)
    for j in range(ROWS_PT // CHUNK):
        pltpu.sync_copy(rows_v,
                        acc.at[pl.ds(sid * ROWS_PT + j * CHUNK, CHUNK)])
    plsc.subcore_barrier()

    def chunk(c, carry):
        pltpu.sync_copy(x_hbm.at[src_v.at[c]], rows_v)
        pltpu.sync_copy(rows_v, acc.at[dst_v.at[c]], add=True)
        return carry

    lax.fori_loop(0, CPT, chunk, 0)
    plsc.subcore_barrier()
    r0 = sid * ROWS_PT
    pltpu.sync_copy(acc.at[pl.ds(r0, ROWS_PT)],
                    out_hbm.at[cid].at[pl.ds(r0, ROWS_PT)])


def _make_spmm(d):
    return functools.partial(
        pl.kernel,
        out_type=jax.ShapeDtypeStruct((2, NP, d), jnp.float32),
        mesh=_mesh,
        compiler_params=_sc_params,
        scratch_types=[
            pltpu.VMEM((CPT, CHUNK), jnp.int32),
            pltpu.VMEM((CPT, CHUNK), jnp.int32),
            pltpu.VMEM((CHUNK, d), jnp.float32),
            pltpu.VMEM_SHARED((NP, d), jnp.float32),
        ],
    )(_spmm_body)


_spmm128 = _make_spmm(128)
_spmm64 = _make_spmm(64)

ROWB = 256


def _norm(d0, d1):
    deg = (d0 + d1)[:, 0:1]
    return lax.rsqrt(jnp.where(deg > 0.0, deg, 1.0))


def _tc_scale_body(f_ref, d0_ref, d1_ref, o_ref):
    o_ref[...] = f_ref[...] * _norm(d0_ref[...], d1_ref[...])


def _tc_mid_body(a0_ref, a1_ref, di0_ref, di1_ref, do0_ref, do1_ref,
                 w1_ref, b1_ref, w2_ref, o_ref):
    ni = _norm(di0_ref[...], di1_ref[...])
    a = (a0_ref[...] + a1_ref[...]) * ni
    h = jnp.dot(a, w1_ref[...], preferred_element_type=jnp.float32)
    h = jnp.maximum(h + b1_ref[...], 0.0)
    no = _norm(do0_ref[...], do1_ref[...])
    y = jnp.dot(h * no, w2_ref[...], preferred_element_type=jnp.float32)
    row = pl.program_id(0) * ROWB + lax.broadcasted_iota(
        jnp.int32, (ROWB, 1), 0)
    o_ref[...] = jnp.where(row < N, y, 0.0)


def _tc_final_body(g0_ref, g1_ref, di0_ref, di1_ref, b2_ref, o_ref):
    ni = _norm(di0_ref[...], di1_ref[...])
    o_ref[...] = (g0_ref[...] + g1_ref[...]) * ni + b2_ref[...]


def _rows_spec(d):
    return pl.BlockSpec((ROWB, d), lambda i: (i, 0))


def _full_spec(shape):
    return pl.BlockSpec(shape, lambda i: tuple(0 for _ in shape))


def kernel(features, edge_index, W1, b1, W2, b2):
    f32 = jnp.float32
    src = edge_index[0]
    dst = edge_index[1]
    pad = jnp.full((EP - E,), PAD_IDX, dtype=jnp.int32)
    src_p = jnp.concatenate([src, pad]).reshape(EP // CHUNK, CHUNK)
    dst_p = jnp.concatenate([dst, pad]).reshape(EP // CHUNK, CHUNK)

    ones_rows = jnp.zeros((2 * CHUNK, DW), f32).at[:CHUNK, 0].set(1.0)
    degp = _deg_kernel(src_p, dst_p, ones_rows)
    dO0, dI0 = degp[0, 0], degp[0, 1]
    dO1, dI1 = degp[1, 0], degp[1, 1]

    feats_p = jnp.pad(features, ((0, NP - N), (0, 0)))
    grid = (NP // ROWB,)
    xs = pl.pallas_call(
        _tc_scale_body,
        grid=grid,
        in_specs=[_rows_spec(128), _rows_spec(DW), _rows_spec(DW)],
        out_specs=_rows_spec(128),
        out_shape=jax.ShapeDtypeStruct((NP, 128), f32),
    )(feats_p, dO0, dO1)

    zeros128 = jnp.zeros((CHUNK, 128), f32)
    agg1 = _spmm128(xs, src_p, dst_p, zeros128)

    y = pl.pallas_call(
        _tc_mid_body,
        grid=grid,
        in_specs=[_rows_spec(128), _rows_spec(128),
                  _rows_spec(DW), _rows_spec(DW),
                  _rows_spec(DW), _rows_spec(DW),
                  _full_spec((128, 256)), _full_spec((1, 256)),
                  _full_spec((256, 64))],
        out_specs=_rows_spec(64),
        out_shape=jax.ShapeDtypeStruct((NP, 64), f32),
    )(agg1[0], agg1[1], dI0, dI1, dO0, dO1,
      W1, b1.reshape(1, 256), W2)

    zeros64 = jnp.zeros((CHUNK, 64), f32)
    agg2 = _spmm64(y, src_p, dst_p, zeros64)

    out = pl.pallas_call(
        _tc_final_body,
        grid=grid,
        in_specs=[_rows_spec(64), _rows_spec(64),
                  _rows_spec(DW), _rows_spec(DW),
                  _full_spec((1, 64))],
        out_specs=_rows_spec(64),
        out_shape=jax.ShapeDtypeStruct((NP, 64), f32),
    )(agg2[0], agg2[1], dI0, dI1, b2.reshape(1, 64))

    return out[:N]


# speedup vs baseline: 3.7104x; 1.0789x over previous
import functools

import jax
import jax.numpy as jnp
from jax import lax
from jax.experimental import pallas as pl
from jax.experimental.pallas import tpu as pltpu
from jax.experimental.pallas import tpu_sc as plsc

N = 10000
NP = 10240
E = 320000
EP = 327680
CHUNK = 128
CPT = EP // (32 * CHUNK)
ROWS_PT = NP // 16
PAD_IDX = NP - 1
DW = 16

_mesh = plsc.VectorSubcoreMesh(core_axis_name="c", subcore_axis_name="s")


def _deg_body(src_hbm, dst_hbm, ones_hbm, out_hbm,
              src_v, dst_v, ones_v, acc_s, acc_d):
    cid = lax.axis_index("c")
    sid = lax.axis_index("s")
    wid = sid * 2 + cid
    pltpu.sync_copy(src_hbm.at[pl.ds(wid * CPT, CPT)], src_v)
    pltpu.sync_copy(dst_hbm.at[pl.ds(wid * CPT, CPT)], dst_v)
    pltpu.sync_copy(ones_hbm, ones_v)
    for j in range(ROWS_PT // CHUNK):
        pltpu.sync_copy(ones_hbm.at[pl.ds(CHUNK, CHUNK)],
                        acc_s.at[pl.ds(sid * ROWS_PT + j * CHUNK, CHUNK)])
        pltpu.sync_copy(ones_hbm.at[pl.ds(CHUNK, CHUNK)],
                        acc_d.at[pl.ds(sid * ROWS_PT + j * CHUNK, CHUNK)])
    plsc.subcore_barrier()

    def chunk(c, carry):
        pltpu.sync_copy(ones_v.at[pl.ds(0, CHUNK)], acc_s.at[src_v.at[c]],
                        add=True)
        pltpu.sync_copy(ones_v.at[pl.ds(0, CHUNK)], acc_d.at[dst_v.at[c]],
                        add=True)
        return carry

    lax.fori_loop(0, CPT, chunk, 0)
    plsc.subcore_barrier()
    r0 = sid * ROWS_PT
    pltpu.sync_copy(acc_s.at[pl.ds(r0, ROWS_PT)],
                    out_hbm.at[cid, 0].at[pl.ds(r0, ROWS_PT)])
    pltpu.sync_copy(acc_d.at[pl.ds(r0, ROWS_PT)],
                    out_hbm.at[cid, 1].at[pl.ds(r0, ROWS_PT)])


_sc_params = pltpu.CompilerParams(use_tc_tiling_on_sc=False)

_deg_kernel = functools.partial(
    pl.kernel,
    out_type=jax.ShapeDtypeStruct((2, 2, NP, DW), jnp.float32),
    mesh=_mesh,
    compiler_params=_sc_params,
    scratch_types=[
        pltpu.VMEM((CPT, CHUNK), jnp.int32),
        pltpu.VMEM((CPT, CHUNK), jnp.int32),
        pltpu.VMEM((2 * CHUNK, DW), jnp.float32),
        pltpu.VMEM_SHARED((NP, DW), jnp.float32),
        pltpu.VMEM_SHARED((NP, DW), jnp.float32),
    ],
)(_deg_body)


def _spmm_body(x_hbm, src_hbm, dst_hbm, zeros_hbm, out_hbm,
               src_v, dst_v, b0, b1, b2, b3, acc,
               g0, g1, g2, g3, s0, s1, s2, s3):
    cid = lax.axis_index("c")
    sid = lax.axis_index("s")
    wid = sid * 2 + cid
    pltpu.sync_copy(src_hbm.at[pl.ds(wid * CPT, CPT)], src_v)
    pltpu.sync_copy(dst_hbm.at[pl.ds(wid * CPT, CPT)], dst_v)
    pltpu.sync_copy(zeros_hbm, b0)
    for j in range(ROWS_PT // CHUNK):
        pltpu.sync_copy(b0,
                        acc.at[pl.ds(sid * ROWS_PT + j * CHUNK, CHUNK)])
    plsc.subcore_barrier()

    bufs = (b0, b1, b2, b3)
    gsems = (g0, g1, g2, g3)
    ssems = (s0, s1, s2, s3)

    def gstart(c, b):
        pltpu.async_copy(x_hbm.at[src_v.at[c]], bufs[b], gsems[b])

    def gwait(c, b):
        pltpu.make_async_copy(x_hbm.at[src_v.at[c]], bufs[b],
                              gsems[b]).wait()

    def sstart(c, b):
        pltpu.async_copy(bufs[b], acc.at[dst_v.at[c]], ssems[b], add=True)

    def swait(c, b):
        pltpu.make_async_copy(bufs[b], acc.at[dst_v.at[c]],
                              ssems[b]).wait()

    gstart(0, 0)
    gstart(1, 1)
    gwait(0, 0)
    sstart(0, 0)
    gwait(1, 1)
    sstart(1, 1)
    gstart(2, 2)
    gstart(3, 3)

    def body(i, carry):
        c = 4 * i
        gwait(c + 2, 2)
        sstart(c + 2, 2)
        gwait(c + 3, 3)
        sstart(c + 3, 3)
        swait(c, 0)
        swait(c + 1, 1)
        gstart(c + 4, 0)
        gstart(c + 5, 1)
        gwait(c + 4, 0)
        sstart(c + 4, 0)
        gwait(c + 5, 1)
        sstart(c + 5, 1)
        swait(c + 2, 2)
        swait(c + 3, 3)
        gstart(c + 6, 2)
        gstart(c + 7, 3)
        return carry

    lax.fori_loop(0, (CPT - 4) // 4, body, 0)
    t = CPT - 4
    gwait(t + 2, 2)
    sstart(t + 2, 2)
    gwait(t + 3, 3)
    sstart(t + 3, 3)
    swait(t, 0)
    swait(t + 1, 1)
    swait(t + 2, 2)
    swait(t + 3, 3)
    plsc.subcore_barrier()
    r0 = sid * ROWS_PT
    pltpu.sync_copy(acc.at[pl.ds(r0, ROWS_PT)],
                    out_hbm.at[cid].at[pl.ds(r0, ROWS_PT)])


SB = 16
NSB = CPT // SB


def _spmm_sb_body(x_hbm, src_hbm, dst_hbm, zeros_hbm, out_hbm,
                  src_v, dst_v, b0, b1, acc, g0, g1, s0, s1):
    cid = lax.axis_index("c")
    sid = lax.axis_index("s")
    wid = sid * 2 + cid
    pltpu.sync_copy(zeros_hbm, b0)
    for j in range(ROWS_PT // CHUNK):
        pltpu.sync_copy(b0,
                        acc.at[pl.ds(sid * ROWS_PT + j * CHUNK, CHUNK)])
    plsc.subcore_barrier()

    bufs = (b0, b1)
    gsems = (g0, g1)
    ssems = (s0, s1)

    def gstart(j, b):
        pltpu.async_copy(x_hbm.at[src_v.at[j]], bufs[b], gsems[b])

    def gwait(j, b):
        pltpu.make_async_copy(x_hbm.at[src_v.at[j]], bufs[b],
                              gsems[b]).wait()

    def sstart(j, b):
        pltpu.async_copy(bufs[b], acc.at[dst_v.at[j]], ssems[b], add=True)

    def swait(j, b):
        pltpu.make_async_copy(bufs[b], acc.at[dst_v.at[j]],
                              ssems[b]).wait()

    def superblock(sb, carry):
        row0 = wid * CPT + sb * SB
        pltpu.sync_copy(src_hbm.at[pl.ds(row0, SB)], src_v)
        pltpu.sync_copy(dst_hbm.at[pl.ds(row0, SB)], dst_v)
        gstart(0, 0)
        for j in range(SB):
            b = j % 2
            gwait(j, b)
            sstart(j, b)
            if j > 0:
                swait(j - 1, 1 - b)
            if j + 1 < SB:
                gstart(j + 1, 1 - b)
        swait(SB - 1, (SB - 1) % 2)
        return carry

    lax.fori_loop(0, NSB, superblock, 0)
    plsc.subcore_barrier()
    r0 = sid * ROWS_PT
    pltpu.sync_copy(acc.at[pl.ds(r0, ROWS_PT)],
                    out_hbm.at[cid].at[pl.ds(r0, ROWS_PT)])


_spmm128 = functools.partial(
    pl.kernel,
    out_type=jax.ShapeDtypeStruct((2, NP, 128), jnp.float32),
    mesh=_mesh,
    compiler_params=_sc_params,
    scratch_types=[
        pltpu.VMEM((SB, CHUNK), jnp.int32),
        pltpu.VMEM((SB, CHUNK), jnp.int32),
        pltpu.VMEM((CHUNK, 128), jnp.float32),
        pltpu.VMEM((CHUNK, 128), jnp.float32),
        pltpu.VMEM_SHARED((NP, 128), jnp.float32),
        pltpu.SemaphoreType.DMA,
        pltpu.SemaphoreType.DMA,
        pltpu.SemaphoreType.DMA,
        pltpu.SemaphoreType.DMA,
    ],
)(_spmm_sb_body)

_spmm64 = functools.partial(
    pl.kernel,
    out_type=jax.ShapeDtypeStruct((2, NP, 64), jnp.float32),
    mesh=_mesh,
    compiler_params=_sc_params,
    scratch_types=[
        pltpu.VMEM((CPT, CHUNK), jnp.int32),
        pltpu.VMEM((CPT, CHUNK), jnp.int32),
        pltpu.VMEM((CHUNK, 64), jnp.float32),
        pltpu.VMEM((CHUNK, 64), jnp.float32),
        pltpu.VMEM((CHUNK, 64), jnp.float32),
        pltpu.VMEM((CHUNK, 64), jnp.float32),
        pltpu.VMEM_SHARED((NP, 64), jnp.float32),
        pltpu.SemaphoreType.DMA,
        pltpu.SemaphoreType.DMA,
        pltpu.SemaphoreType.DMA,
        pltpu.SemaphoreType.DMA,
        pltpu.SemaphoreType.DMA,
        pltpu.SemaphoreType.DMA,
        pltpu.SemaphoreType.DMA,
        pltpu.SemaphoreType.DMA,
    ],
)(_spmm_body)


ROWB = 256


def _norm(d0, d1):
    deg = (d0 + d1)[:, 0:1]
    return lax.rsqrt(jnp.where(deg > 0.0, deg, 1.0))


def _tc_scale_body(f_ref, d0_ref, d1_ref, o_ref):
    o_ref[...] = f_ref[...] * _norm(d0_ref[...], d1_ref[...])


def _tc_mid_body(a0_ref, a1_ref, di0_ref, di1_ref, do0_ref, do1_ref,
                 w1_ref, b1_ref, w2_ref, o_ref):
    ni = _norm(di0_ref[...], di1_ref[...])
    a = (a0_ref[...] + a1_ref[...]) * ni
    h = jnp.dot(a, w1_ref[...], preferred_element_type=jnp.float32)
    h = jnp.maximum(h + b1_ref[...], 0.0)
    no = _norm(do0_ref[...], do1_ref[...])
    y = jnp.dot(h * no, w2_ref[...], preferred_element_type=jnp.float32)
    row = pl.program_id(0) * ROWB + lax.broadcasted_iota(
        jnp.int32, (ROWB, 1), 0)
    o_ref[...] = jnp.where(row < N, y, 0.0)


def _tc_final_body(g0_ref, g1_ref, di0_ref, di1_ref, b2_ref, o_ref):
    ni = _norm(di0_ref[...], di1_ref[...])
    o_ref[...] = (g0_ref[...] + g1_ref[...]) * ni + b2_ref[...]


def _rows_spec(d):
    return pl.BlockSpec((ROWB, d), lambda i: (i, 0))


def _full_spec(shape):
    return pl.BlockSpec(shape, lambda i: tuple(0 for _ in shape))


def kernel(features, edge_index, W1, b1, W2, b2):
    f32 = jnp.float32
    src = edge_index[0]
    dst = edge_index[1]
    pad = jnp.full((EP - E,), PAD_IDX, dtype=jnp.int32)
    src_p = jnp.concatenate([src, pad]).reshape(EP // CHUNK, CHUNK)
    dst_p = jnp.concatenate([dst, pad]).reshape(EP // CHUNK, CHUNK)

    ones_rows = jnp.zeros((2 * CHUNK, DW), f32).at[:CHUNK, 0].set(1.0)
    degp = _deg_kernel(src_p, dst_p, ones_rows)
    dO0, dI0 = degp[0, 0], degp[0, 1]
    dO1, dI1 = degp[1, 0], degp[1, 1]

    feats_p = jnp.pad(features, ((0, NP - N), (0, 0)))
    grid = (NP // ROWB,)
    xs = pl.pallas_call(
        _tc_scale_body,
        grid=grid,
        in_specs=[_rows_spec(128), _rows_spec(DW), _rows_spec(DW)],
        out_specs=_rows_spec(128),
        out_shape=jax.ShapeDtypeStruct((NP, 128), f32),
    )(feats_p, dO0, dO1)

    zeros128 = jnp.zeros((CHUNK, 128), f32)
    agg1 = _spmm128(xs, src_p, dst_p, zeros128)

    y = pl.pallas_call(
        _tc_mid_body,
        grid=grid,
        in_specs=[_rows_spec(128), _rows_spec(128),
                  _rows_spec(DW), _rows_spec(DW),
                  _rows_spec(DW), _rows_spec(DW),
                  _full_spec((128, 256)), _full_spec((1, 256)),
                  _full_spec((256, 64))],
        out_specs=_rows_spec(64),
        out_shape=jax.ShapeDtypeStruct((NP, 64), f32),
    )(agg1[0], agg1[1], dI0, dI1, dO0, dO1,
      W1, b1.reshape(1, 256), W2)

    zeros64 = jnp.zeros((CHUNK, 64), f32)
    agg2 = _spmm64(y, src_p, dst_p, zeros64)

    out = pl.pallas_call(
        _tc_final_body,
        grid=grid,
        in_specs=[_rows_spec(64), _rows_spec(64),
                  _rows_spec(DW), _rows_spec(DW),
                  _full_spec((1, 64))],
        out_specs=_rows_spec(64),
        out_shape=jax.ShapeDtypeStruct((NP, 64), f32),
    )(agg2[0], agg2[1], dI0, dI1, b2.reshape(1, 64))

    return out[:N]


# speedup vs baseline: 4.4780x; 1.2069x over previous
import functools

import jax
import jax.numpy as jnp
from jax import lax
from jax.experimental import pallas as pl
from jax.experimental.pallas import tpu as pltpu
from jax.experimental.pallas import tpu_sc as plsc

N = 10000
NP = 10240
E = 320000
EP = 327680
CHUNK = 128
CPT = EP // (32 * CHUNK)
ROWS_PT = NP // 16
PAD_IDX = NP - 1
DW = 8

_mesh = plsc.VectorSubcoreMesh(core_axis_name="c", subcore_axis_name="s")


def _deg_body(src_hbm, dst_hbm, ones_hbm, out_hbm,
              src_v, dst_v, ones_v, acc_s, acc_d, m0, m1, m2, m3):
    cid = lax.axis_index("c")
    sid = lax.axis_index("s")
    wid = sid * 2 + cid
    pltpu.sync_copy(src_hbm.at[pl.ds(wid * CPT, CPT)], src_v)
    pltpu.sync_copy(dst_hbm.at[pl.ds(wid * CPT, CPT)], dst_v)
    pltpu.sync_copy(ones_hbm, ones_v)
    for j in range(ROWS_PT // CHUNK):
        pltpu.sync_copy(ones_hbm.at[pl.ds(CHUNK, CHUNK)],
                        acc_s.at[pl.ds(sid * ROWS_PT + j * CHUNK, CHUNK)])
        pltpu.sync_copy(ones_hbm.at[pl.ds(CHUNK, CHUNK)],
                        acc_d.at[pl.ds(sid * ROWS_PT + j * CHUNK, CHUNK)])
    plsc.subcore_barrier()

    sems = (m0, m1, m2, m3)
    ones_row = ones_v.at[pl.ds(0, CHUNK)]

    def sadd(idx_row, acc, sem):
        pltpu.async_copy(ones_row, acc.at[idx_row], sem, add=True)

    def sdrain(idx_row, acc, sem):
        pltpu.make_async_copy(ones_row, acc.at[idx_row], sem).wait()

    del sems
    sadd(src_v.at[0], acc_s, m0)
    sadd(dst_v.at[0], acc_d, m1)
    sadd(src_v.at[1], acc_s, m2)
    sadd(dst_v.at[1], acc_d, m3)

    def chunk(i, carry):
        c = 2 * i
        sdrain(src_v.at[c], acc_s, m0)
        sdrain(dst_v.at[c], acc_d, m1)
        sadd(src_v.at[c + 2], acc_s, m0)
        sadd(dst_v.at[c + 2], acc_d, m1)
        sdrain(src_v.at[c + 1], acc_s, m2)
        sdrain(dst_v.at[c + 1], acc_d, m3)
        sadd(src_v.at[c + 3], acc_s, m2)
        sadd(dst_v.at[c + 3], acc_d, m3)
        return carry

    lax.fori_loop(0, (CPT - 2) // 2, chunk, 0)
    t = CPT - 2
    sdrain(src_v.at[t], acc_s, m0)
    sdrain(dst_v.at[t], acc_d, m1)
    sdrain(src_v.at[t + 1], acc_s, m2)
    sdrain(dst_v.at[t + 1], acc_d, m3)
    plsc.subcore_barrier()
    r0 = sid * ROWS_PT
    pltpu.sync_copy(acc_s.at[pl.ds(r0, ROWS_PT)],
                    out_hbm.at[cid, 0].at[pl.ds(r0, ROWS_PT)])
    pltpu.sync_copy(acc_d.at[pl.ds(r0, ROWS_PT)],
                    out_hbm.at[cid, 1].at[pl.ds(r0, ROWS_PT)])


_sc_params = pltpu.CompilerParams(use_tc_tiling_on_sc=False)

_deg_kernel = functools.partial(
    pl.kernel,
    out_type=jax.ShapeDtypeStruct((2, 2, NP, DW), jnp.float32),
    mesh=_mesh,
    compiler_params=_sc_params,
    scratch_types=[
        pltpu.VMEM((CPT, CHUNK), jnp.int32),
        pltpu.VMEM((CPT, CHUNK), jnp.int32),
        pltpu.VMEM((2 * CHUNK, DW), jnp.float32),
        pltpu.VMEM_SHARED((NP, DW), jnp.float32),
        pltpu.VMEM_SHARED((NP, DW), jnp.float32),
        pltpu.SemaphoreType.DMA,
        pltpu.SemaphoreType.DMA,
        pltpu.SemaphoreType.DMA,
        pltpu.SemaphoreType.DMA,
    ],
)(_deg_body)


def _spmm_body(x_hbm, src_hbm, dst_hbm, zeros_hbm, out_hbm,
               src_v, dst_v, b0, b1, b2, b3, acc,
               g0, g1, g2, g3, s0, s1, s2, s3):
    cid = lax.axis_index("c")
    sid = lax.axis_index("s")
    wid = sid * 2 + cid
    pltpu.sync_copy(src_hbm.at[pl.ds(wid * CPT, CPT)], src_v)
    pltpu.sync_copy(dst_hbm.at[pl.ds(wid * CPT, CPT)], dst_v)
    pltpu.sync_copy(zeros_hbm, b0)
    for j in range(ROWS_PT // CHUNK):
        pltpu.sync_copy(b0,
                        acc.at[pl.ds(sid * ROWS_PT + j * CHUNK, CHUNK)])
    plsc.subcore_barrier()

    bufs = (b0, b1, b2, b3)
    gsems = (g0, g1, g2, g3)
    ssems = (s0, s1, s2, s3)

    def gstart(c, b):
        pltpu.async_copy(x_hbm.at[src_v.at[c]], bufs[b], gsems[b])

    def gwait(c, b):
        pltpu.make_async_copy(x_hbm.at[src_v.at[c]], bufs[b],
                              gsems[b]).wait()

    def sstart(c, b):
        pltpu.async_copy(bufs[b], acc.at[dst_v.at[c]], ssems[b], add=True)

    def swait(c, b):
        pltpu.make_async_copy(bufs[b], acc.at[dst_v.at[c]],
                              ssems[b]).wait()

    gstart(0, 0)
    gstart(1, 1)
    gwait(0, 0)
    sstart(0, 0)
    gwait(1, 1)
    sstart(1, 1)
    gstart(2, 2)
    gstart(3, 3)

    def body(i, carry):
        c = 4 * i
        gwait(c + 2, 2)
        sstart(c + 2, 2)
        gwait(c + 3, 3)
        sstart(c + 3, 3)
        swait(c, 0)
        swait(c + 1, 1)
        gstart(c + 4, 0)
        gstart(c + 5, 1)
        gwait(c + 4, 0)
        sstart(c + 4, 0)
        gwait(c + 5, 1)
        sstart(c + 5, 1)
        swait(c + 2, 2)
        swait(c + 3, 3)
        gstart(c + 6, 2)
        gstart(c + 7, 3)
        return carry

    lax.fori_loop(0, (CPT - 4) // 4, body, 0)
    t = CPT - 4
    gwait(t + 2, 2)
    sstart(t + 2, 2)
    gwait(t + 3, 3)
    sstart(t + 3, 3)
    swait(t, 0)
    swait(t + 1, 1)
    swait(t + 2, 2)
    swait(t + 3, 3)
    plsc.subcore_barrier()
    r0 = sid * ROWS_PT
    pltpu.sync_copy(acc.at[pl.ds(r0, ROWS_PT)],
                    out_hbm.at[cid].at[pl.ds(r0, ROWS_PT)])


SB = 16
NSB = CPT // SB


def _spmm_sb_body(x_hbm, src_hbm, dst_hbm, zeros_hbm, out_hbm,
                  src_v, dst_v, b0, b1, acc, g0, g1, s0, s1):
    cid = lax.axis_index("c")
    sid = lax.axis_index("s")
    wid = sid * 2 + cid
    pltpu.sync_copy(zeros_hbm, b0)
    for j in range(ROWS_PT // CHUNK):
        pltpu.sync_copy(b0,
                        acc.at[pl.ds(sid * ROWS_PT + j * CHUNK, CHUNK)])
    plsc.subcore_barrier()

    bufs = (b0, b1)
    gsems = (g0, g1)
    ssems = (s0, s1)

    def gstart(j, b):
        pltpu.async_copy(x_hbm.at[src_v.at[j]], bufs[b], gsems[b])

    def gwait(j, b):
        pltpu.make_async_copy(x_hbm.at[src_v.at[j]], bufs[b],
                              gsems[b]).wait()

    def sstart(j, b):
        pltpu.async_copy(bufs[b], acc.at[dst_v.at[j]], ssems[b], add=True)

    def swait(j, b):
        pltpu.make_async_copy(bufs[b], acc.at[dst_v.at[j]],
                              ssems[b]).wait()

    def superblock(sb, carry):
        row0 = wid * CPT + sb * SB
        pltpu.sync_copy(src_hbm.at[pl.ds(row0, SB)], src_v)
        pltpu.sync_copy(dst_hbm.at[pl.ds(row0, SB)], dst_v)
        gstart(0, 0)
        for j in range(SB):
            b = j % 2
            gwait(j, b)
            sstart(j, b)
            if j > 0:
                swait(j - 1, 1 - b)
            if j + 1 < SB:
                gstart(j + 1, 1 - b)
        swait(SB - 1, (SB - 1) % 2)
        return carry

    lax.fori_loop(0, NSB, superblock, 0)
    plsc.subcore_barrier()
    r0 = sid * ROWS_PT
    pltpu.sync_copy(acc.at[pl.ds(r0, ROWS_PT)],
                    out_hbm.at[cid].at[pl.ds(r0, ROWS_PT)])


_spmm128 = functools.partial(
    pl.kernel,
    out_type=jax.ShapeDtypeStruct((2, NP, 128), jnp.float32),
    mesh=_mesh,
    compiler_params=_sc_params,
    scratch_types=[
        pltpu.VMEM((SB, CHUNK), jnp.int32),
        pltpu.VMEM((SB, CHUNK), jnp.int32),
        pltpu.VMEM((CHUNK, 128), jnp.float32),
        pltpu.VMEM((CHUNK, 128), jnp.float32),
        pltpu.VMEM_SHARED((NP, 128), jnp.float32),
        pltpu.SemaphoreType.DMA,
        pltpu.SemaphoreType.DMA,
        pltpu.SemaphoreType.DMA,
        pltpu.SemaphoreType.DMA,
    ],
)(_spmm_sb_body)

_spmm64 = functools.partial(
    pl.kernel,
    out_type=jax.ShapeDtypeStruct((2, NP, 64), jnp.float32),
    mesh=_mesh,
    compiler_params=_sc_params,
    scratch_types=[
        pltpu.VMEM((CPT, CHUNK), jnp.int32),
        pltpu.VMEM((CPT, CHUNK), jnp.int32),
        pltpu.VMEM((CHUNK, 64), jnp.float32),
        pltpu.VMEM((CHUNK, 64), jnp.float32),
        pltpu.VMEM((CHUNK, 64), jnp.float32),
        pltpu.VMEM((CHUNK, 64), jnp.float32),
        pltpu.VMEM_SHARED((NP, 64), jnp.float32),
        pltpu.SemaphoreType.DMA,
        pltpu.SemaphoreType.DMA,
        pltpu.SemaphoreType.DMA,
        pltpu.SemaphoreType.DMA,
        pltpu.SemaphoreType.DMA,
        pltpu.SemaphoreType.DMA,
        pltpu.SemaphoreType.DMA,
        pltpu.SemaphoreType.DMA,
    ],
)(_spmm_body)


ROWB = 256


def _norm(d0, d1):
    deg = (d0 + d1)[:, 0:1]
    return lax.rsqrt(jnp.where(deg > 0.0, deg, 1.0))


def _tc_scale_body(f_ref, d0_ref, d1_ref, o_ref):
    o_ref[...] = f_ref[...] * _norm(d0_ref[...], d1_ref[...])


def _tc_mid_body(a0_ref, a1_ref, di0_ref, di1_ref, do0_ref, do1_ref,
                 w1_ref, b1_ref, w2_ref, o_ref):
    ni = _norm(di0_ref[...], di1_ref[...])
    a = (a0_ref[...] + a1_ref[...]) * ni
    h = jnp.dot(a, w1_ref[...], preferred_element_type=jnp.float32)
    h = jnp.maximum(h + b1_ref[...], 0.0)
    no = _norm(do0_ref[...], do1_ref[...])
    y = jnp.dot(h * no, w2_ref[...], preferred_element_type=jnp.float32)
    row = pl.program_id(0) * ROWB + lax.broadcasted_iota(
        jnp.int32, (ROWB, 1), 0)
    o_ref[...] = jnp.where(row < N, y, 0.0)


def _tc_final_body(g0_ref, g1_ref, di0_ref, di1_ref, b2_ref, o_ref):
    ni = _norm(di0_ref[...], di1_ref[...])
    o_ref[...] = (g0_ref[...] + g1_ref[...]) * ni + b2_ref[...]


def _rows_spec(d):
    return pl.BlockSpec((ROWB, d), lambda i: (i, 0))


def _full_spec(shape):
    return pl.BlockSpec(shape, lambda i: tuple(0 for _ in shape))


def kernel(features, edge_index, W1, b1, W2, b2):
    f32 = jnp.float32
    src = edge_index[0]
    dst = edge_index[1]
    pad = jnp.full((EP - E,), PAD_IDX, dtype=jnp.int32)
    src_p = jnp.concatenate([src, pad]).reshape(EP // CHUNK, CHUNK)
    dst_p = jnp.concatenate([dst, pad]).reshape(EP // CHUNK, CHUNK)

    ones_rows = jnp.zeros((2 * CHUNK, DW), f32).at[:CHUNK, 0].set(1.0)
    degp = _deg_kernel(src_p, dst_p, ones_rows)
    dO0, dI0 = degp[0, 0], degp[0, 1]
    dO1, dI1 = degp[1, 0], degp[1, 1]

    feats_p = jnp.pad(features, ((0, NP - N), (0, 0)))
    grid = (NP // ROWB,)
    xs = pl.pallas_call(
        _tc_scale_body,
        grid=grid,
        in_specs=[_rows_spec(128), _rows_spec(DW), _rows_spec(DW)],
        out_specs=_rows_spec(128),
        out_shape=jax.ShapeDtypeStruct((NP, 128), f32),
    )(feats_p, dO0, dO1)

    zeros128 = jnp.zeros((CHUNK, 128), f32)
    agg1 = _spmm128(xs, src_p, dst_p, zeros128)

    y = pl.pallas_call(
        _tc_mid_body,
        grid=grid,
        in_specs=[_rows_spec(128), _rows_spec(128),
                  _rows_spec(DW), _rows_spec(DW),
                  _rows_spec(DW), _rows_spec(DW),
                  _full_spec((128, 256)), _full_spec((1, 256)),
                  _full_spec((256, 64))],
        out_specs=_rows_spec(64),
        out_shape=jax.ShapeDtypeStruct((NP, 64), f32),
    )(agg1[0], agg1[1], dI0, dI1, dO0, dO1,
      W1, b1.reshape(1, 256), W2)

    zeros64 = jnp.zeros((CHUNK, 64), f32)
    agg2 = _spmm64(y, src_p, dst_p, zeros64)

    out = pl.pallas_call(
        _tc_final_body,
        grid=grid,
        in_specs=[_rows_spec(64), _rows_spec(64),
                  _rows_spec(DW), _rows_spec(DW),
                  _full_spec((1, 64))],
        out_specs=_rows_spec(64),
        out_shape=jax.ShapeDtypeStruct((NP, 64), f32),
    )(agg2[0], agg2[1], dI0, dI1, b2.reshape(1, 64))

    return out[:N]


# speedup vs baseline: 4.6397x; 1.0361x over previous
import functools

import jax
import jax.numpy as jnp
from jax import lax
from jax.experimental import pallas as pl
from jax.experimental.pallas import tpu as pltpu
from jax.experimental.pallas import tpu_sc as plsc

N = 10000
NP = 10240
E = 320000
EP = 327680
CHUNK = 128
CPT = EP // (32 * CHUNK)
ROWS_PT = NP // 16
PAD_IDX = NP - 1
DW = 8

_mesh = plsc.VectorSubcoreMesh(core_axis_name="c", subcore_axis_name="s")


def _deg_body(src_hbm, dst_hbm, ones_hbm, out_hbm,
              src_v, dst_v, ones_v, acc_s, acc_d, m0, m1, m2, m3):
    cid = lax.axis_index("c")
    sid = lax.axis_index("s")
    wid = sid * 2 + cid
    pltpu.sync_copy(src_hbm.at[pl.ds(wid * CPT, CPT)], src_v)
    pltpu.sync_copy(dst_hbm.at[pl.ds(wid * CPT, CPT)], dst_v)
    pltpu.sync_copy(ones_hbm, ones_v)
    for j in range(ROWS_PT // CHUNK):
        pltpu.sync_copy(ones_hbm.at[pl.ds(CHUNK, CHUNK)],
                        acc_s.at[pl.ds(sid * ROWS_PT + j * CHUNK, CHUNK)])
        pltpu.sync_copy(ones_hbm.at[pl.ds(CHUNK, CHUNK)],
                        acc_d.at[pl.ds(sid * ROWS_PT + j * CHUNK, CHUNK)])
    plsc.subcore_barrier()

    sems = (m0, m1, m2, m3)
    ones_row = ones_v.at[pl.ds(0, CHUNK)]

    def sadd(idx_row, acc, sem):
        pltpu.async_copy(ones_row, acc.at[idx_row], sem, add=True)

    def sdrain(idx_row, acc, sem):
        pltpu.make_async_copy(ones_row, acc.at[idx_row], sem).wait()

    del sems
    sadd(src_v.at[0], acc_s, m0)
    sadd(dst_v.at[0], acc_d, m1)
    sadd(src_v.at[1], acc_s, m2)
    sadd(dst_v.at[1], acc_d, m3)

    def chunk(i, carry):
        c = 2 * i
        sdrain(src_v.at[c], acc_s, m0)
        sdrain(dst_v.at[c], acc_d, m1)
        sadd(src_v.at[c + 2], acc_s, m0)
        sadd(dst_v.at[c + 2], acc_d, m1)
        sdrain(src_v.at[c + 1], acc_s, m2)
        sdrain(dst_v.at[c + 1], acc_d, m3)
        sadd(src_v.at[c + 3], acc_s, m2)
        sadd(dst_v.at[c + 3], acc_d, m3)
        return carry

    lax.fori_loop(0, (CPT - 2) // 2, chunk, 0)
    t = CPT - 2
    sdrain(src_v.at[t], acc_s, m0)
    sdrain(dst_v.at[t], acc_d, m1)
    sdrain(src_v.at[t + 1], acc_s, m2)
    sdrain(dst_v.at[t + 1], acc_d, m3)
    plsc.subcore_barrier()
    r0 = sid * ROWS_PT
    pltpu.sync_copy(acc_s.at[pl.ds(r0, ROWS_PT)],
                    out_hbm.at[cid, 0].at[pl.ds(r0, ROWS_PT)])
    pltpu.sync_copy(acc_d.at[pl.ds(r0, ROWS_PT)],
                    out_hbm.at[cid, 1].at[pl.ds(r0, ROWS_PT)])


_sc_params = pltpu.CompilerParams(use_tc_tiling_on_sc=False)

_deg_kernel = functools.partial(
    pl.kernel,
    out_type=jax.ShapeDtypeStruct((2, 2, NP, DW), jnp.float32),
    mesh=_mesh,
    compiler_params=_sc_params,
    scratch_types=[
        pltpu.VMEM((CPT, CHUNK), jnp.int32),
        pltpu.VMEM((CPT, CHUNK), jnp.int32),
        pltpu.VMEM((2 * CHUNK, DW), jnp.float32),
        pltpu.VMEM_SHARED((NP, DW), jnp.float32),
        pltpu.VMEM_SHARED((NP, DW), jnp.float32),
        pltpu.SemaphoreType.DMA,
        pltpu.SemaphoreType.DMA,
        pltpu.SemaphoreType.DMA,
        pltpu.SemaphoreType.DMA,
    ],
)(_deg_body)


EPT = EP // 32


def _make_spmm(d, chunk, nbuf, k, sb):
    nc = EPT // chunk
    nt = nc // sb

    def body(*refs):
        (x_hbm, src_hbm, dst_hbm, zeros_hbm, out_hbm) = refs[:5]
        bufs = refs[5:5 + nbuf]
        src_v = refs[5 + nbuf:7 + nbuf]
        dst_v = refs[7 + nbuf:9 + nbuf]
        acc = refs[9 + nbuf]
        gsems = refs[10 + nbuf:10 + 2 * nbuf]
        ssems = refs[10 + 2 * nbuf:10 + 3 * nbuf]
        isems = refs[10 + 3 * nbuf:12 + 3 * nbuf]

        cid = lax.axis_index("c")
        sid = lax.axis_index("s")
        wid = sid * 2 + cid
        base = wid * nc

        def ifire(t):
            pltpu.async_copy(src_hbm.at[pl.ds(base + t * sb, sb)],
                             src_v[t % 2], isems[0])
            pltpu.async_copy(dst_hbm.at[pl.ds(base + t * sb, sb)],
                             dst_v[t % 2], isems[1])

        def iwait(t):
            pltpu.make_async_copy(src_hbm.at[pl.ds(base + t * sb, sb)],
                                  src_v[t % 2], isems[0]).wait()
            pltpu.make_async_copy(dst_hbm.at[pl.ds(base + t * sb, sb)],
                                  dst_v[t % 2], isems[1]).wait()

        def gstart(c):
            b = c % nbuf
            pltpu.async_copy(x_hbm.at[src_v[(c // sb) % 2].at[c % sb]],
                             bufs[b], gsems[b])

        def gwait(c):
            b = c % nbuf
            pltpu.make_async_copy(
                x_hbm.at[src_v[(c // sb) % 2].at[c % sb]],
                bufs[b], gsems[b]).wait()

        def sstart(c):
            b = c % nbuf
            pltpu.async_copy(bufs[b],
                             acc.at[dst_v[(c // sb) % 2].at[c % sb]],
                             ssems[b], add=True)

        def swait(c):
            b = c % nbuf
            pltpu.make_async_copy(
                bufs[b], acc.at[dst_v[(c // sb) % 2].at[c % sb]],
                ssems[b]).wait()

        ifire(0)
        iwait(0)
        if nt > 1:
            ifire(1)
        for c in range(nbuf):
            gstart(c)
        for j in range(ROWS_PT // 128):
            pltpu.sync_copy(zeros_hbm,
                            acc.at[pl.ds(sid * ROWS_PT + j * 128, 128)])
        plsc.subcore_barrier()
        if nt > 1:
            iwait(1)

        for j in range(nc):
            gwait(j)
            sstart(j)
            i = j - k
            if 0 <= i and i + nbuf < nc:
                swait(i)
                gstart(i + nbuf)
            if j >= k and (j - k + 1) % sb == 0:
                t = (j - k + 1) // sb + 1
                if t < nt:
                    ifire(t)
                    iwait(t)
        for i in range(nc - nbuf, nc):
            swait(i)
        plsc.subcore_barrier()
        r0 = sid * ROWS_PT
        pltpu.sync_copy(acc.at[pl.ds(r0, ROWS_PT)],
                        out_hbm.at[cid].at[pl.ds(r0, ROWS_PT)])

    scratch = (
        [pltpu.VMEM((chunk, d), jnp.float32)] * nbuf
        + [pltpu.VMEM((sb, chunk), jnp.int32)] * 4
        + [pltpu.VMEM_SHARED((NP, d), jnp.float32)]
        + [pltpu.SemaphoreType.DMA] * (2 * nbuf + 2)
    )
    return functools.partial(
        pl.kernel,
        out_type=jax.ShapeDtypeStruct((2, NP, d), jnp.float32),
        mesh=_mesh,
        compiler_params=_sc_params,
        scratch_types=scratch,
    )(body)


_spmm128 = _make_spmm(128, 64, 4, 2, 16)
_spmm64 = _make_spmm(64, 128, 8, 4, 16)


ROWB = 256


def _norm(d0, d1):
    deg = (d0 + d1)[:, 0:1]
    return lax.rsqrt(jnp.where(deg > 0.0, deg, 1.0))


def _tc_scale_body(f_ref, d0_ref, d1_ref, o_ref):
    o_ref[...] = f_ref[...] * _norm(d0_ref[...], d1_ref[...])


def _tc_mid_body(a0_ref, a1_ref, di0_ref, di1_ref, do0_ref, do1_ref,
                 w1_ref, b1_ref, w2_ref, o_ref):
    ni = _norm(di0_ref[...], di1_ref[...])
    a = (a0_ref[...] + a1_ref[...]) * ni
    h = jnp.dot(a, w1_ref[...], preferred_element_type=jnp.float32)
    h = jnp.maximum(h + b1_ref[...], 0.0)
    no = _norm(do0_ref[...], do1_ref[...])
    y = jnp.dot(h * no, w2_ref[...], preferred_element_type=jnp.float32)
    row = pl.program_id(0) * ROWB + lax.broadcasted_iota(
        jnp.int32, (ROWB, 1), 0)
    o_ref[...] = jnp.where(row < N, y, 0.0)


def _tc_final_body(g0_ref, g1_ref, di0_ref, di1_ref, b2_ref, o_ref):
    ni = _norm(di0_ref[...], di1_ref[...])
    o_ref[...] = (g0_ref[...] + g1_ref[...]) * ni + b2_ref[...]


def _rows_spec(d):
    return pl.BlockSpec((ROWB, d), lambda i: (i, 0))


def _full_spec(shape):
    return pl.BlockSpec(shape, lambda i: tuple(0 for _ in shape))


def kernel(features, edge_index, W1, b1, W2, b2):
    f32 = jnp.float32
    src = edge_index[0]
    dst = edge_index[1]
    pad = jnp.full((EP - E,), PAD_IDX, dtype=jnp.int32)
    src_f = jnp.concatenate([src, pad])
    dst_f = jnp.concatenate([dst, pad])
    src_p = src_f.reshape(EP // CHUNK, CHUNK)
    dst_p = dst_f.reshape(EP // CHUNK, CHUNK)
    src_p64 = src_f.reshape(EP // 64, 64)
    dst_p64 = dst_f.reshape(EP // 64, 64)

    ones_rows = jnp.zeros((2 * CHUNK, DW), f32).at[:CHUNK, 0].set(1.0)
    degp = _deg_kernel(src_p, dst_p, ones_rows)
    dO0, dI0 = degp[0, 0], degp[0, 1]
    dO1, dI1 = degp[1, 0], degp[1, 1]

    feats_p = jnp.pad(features, ((0, NP - N), (0, 0)))
    grid = (NP // ROWB,)
    xs = pl.pallas_call(
        _tc_scale_body,
        grid=grid,
        in_specs=[_rows_spec(128), _rows_spec(DW), _rows_spec(DW)],
        out_specs=_rows_spec(128),
        out_shape=jax.ShapeDtypeStruct((NP, 128), f32),
    )(feats_p, dO0, dO1)

    zeros128 = jnp.zeros((128, 128), f32)
    agg1 = _spmm128(xs, src_p64, dst_p64, zeros128)

    y = pl.pallas_call(
        _tc_mid_body,
        grid=grid,
        in_specs=[_rows_spec(128), _rows_spec(128),
                  _rows_spec(DW), _rows_spec(DW),
                  _rows_spec(DW), _rows_spec(DW),
                  _full_spec((128, 256)), _full_spec((1, 256)),
                  _full_spec((256, 64))],
        out_specs=_rows_spec(64),
        out_shape=jax.ShapeDtypeStruct((NP, 64), f32),
    )(agg1[0], agg1[1], dI0, dI1, dO0, dO1,
      W1, b1.reshape(1, 256), W2)

    zeros64 = jnp.zeros((128, 64), f32)
    agg2 = _spmm64(y, src_p, dst_p, zeros64)

    out = pl.pallas_call(
        _tc_final_body,
        grid=grid,
        in_specs=[_rows_spec(64), _rows_spec(64),
                  _rows_spec(DW), _rows_spec(DW),
                  _full_spec((1, 64))],
        out_specs=_rows_spec(64),
        out_shape=jax.ShapeDtypeStruct((NP, 64), f32),
    )(agg2[0], agg2[1], dI0, dI1, b2.reshape(1, 64))

    return out[:N]


# speedup vs baseline: 5.9507x; 1.2826x over previous
import functools

import jax
import jax.numpy as jnp
from jax import lax
from jax.experimental import pallas as pl
from jax.experimental.pallas import tpu as pltpu
from jax.experimental.pallas import tpu_sc as plsc

N = 10000
NP = 10240
E = 320000
EP = 327680
CHUNK = 128
CPT = EP // (32 * CHUNK)
ROWS_PT = NP // 16
PAD_IDX = NP - 1
DW = 8

_mesh = plsc.VectorSubcoreMesh(core_axis_name="c", subcore_axis_name="s")


def _deg_body(src_hbm, dst_hbm, ones_hbm, out_hbm,
              src_v, dst_v, ones_v, acc_s, acc_d, m0, m1, m2, m3):
    cid = lax.axis_index("c")
    sid = lax.axis_index("s")
    wid = sid * 2 + cid
    pltpu.sync_copy(src_hbm.at[pl.ds(wid * CPT, CPT)], src_v)
    pltpu.sync_copy(dst_hbm.at[pl.ds(wid * CPT, CPT)], dst_v)
    pltpu.sync_copy(ones_hbm, ones_v)
    for j in range(ROWS_PT // CHUNK):
        pltpu.sync_copy(ones_hbm.at[pl.ds(CHUNK, CHUNK)],
                        acc_s.at[pl.ds(sid * ROWS_PT + j * CHUNK, CHUNK)])
        pltpu.sync_copy(ones_hbm.at[pl.ds(CHUNK, CHUNK)],
                        acc_d.at[pl.ds(sid * ROWS_PT + j * CHUNK, CHUNK)])
    plsc.subcore_barrier()

    sems = (m0, m1, m2, m3)
    ones_row = ones_v.at[pl.ds(0, CHUNK)]

    def sadd(idx_row, acc, sem):
        pltpu.async_copy(ones_row, acc.at[idx_row], sem, add=True)

    def sdrain(idx_row, acc, sem):
        pltpu.make_async_copy(ones_row, acc.at[idx_row], sem).wait()

    del sems
    sadd(src_v.at[0], acc_s, m0)
    sadd(dst_v.at[0], acc_d, m1)
    sadd(src_v.at[1], acc_s, m2)
    sadd(dst_v.at[1], acc_d, m3)

    def chunk(i, carry):
        c = 2 * i
        sdrain(src_v.at[c], acc_s, m0)
        sdrain(dst_v.at[c], acc_d, m1)
        sadd(src_v.at[c + 2], acc_s, m0)
        sadd(dst_v.at[c + 2], acc_d, m1)
        sdrain(src_v.at[c + 1], acc_s, m2)
        sdrain(dst_v.at[c + 1], acc_d, m3)
        sadd(src_v.at[c + 3], acc_s, m2)
        sadd(dst_v.at[c + 3], acc_d, m3)
        return carry

    lax.fori_loop(0, (CPT - 2) // 2, chunk, 0)
    t = CPT - 2
    sdrain(src_v.at[t], acc_s, m0)
    sdrain(dst_v.at[t], acc_d, m1)
    sdrain(src_v.at[t + 1], acc_s, m2)
    sdrain(dst_v.at[t + 1], acc_d, m3)
    plsc.subcore_barrier()
    r0 = sid * ROWS_PT
    pltpu.sync_copy(acc_s.at[pl.ds(r0, ROWS_PT)],
                    out_hbm.at[cid, 0].at[pl.ds(r0, ROWS_PT)])
    pltpu.sync_copy(acc_d.at[pl.ds(r0, ROWS_PT)],
                    out_hbm.at[cid, 1].at[pl.ds(r0, ROWS_PT)])


_sc_params = pltpu.CompilerParams(use_tc_tiling_on_sc=False)

_deg_kernel = functools.partial(
    pl.kernel,
    out_type=jax.ShapeDtypeStruct((2, 2, NP, DW), jnp.float32),
    mesh=_mesh,
    compiler_params=_sc_params,
    scratch_types=[
        pltpu.VMEM((CPT, CHUNK), jnp.int32),
        pltpu.VMEM((CPT, CHUNK), jnp.int32),
        pltpu.VMEM((2 * CHUNK, DW), jnp.float32),
        pltpu.VMEM_SHARED((NP, DW), jnp.float32),
        pltpu.VMEM_SHARED((NP, DW), jnp.float32),
        pltpu.SemaphoreType.DMA,
        pltpu.SemaphoreType.DMA,
        pltpu.SemaphoreType.DMA,
        pltpu.SemaphoreType.DMA,
    ],
)(_deg_body)


EPT = EP // 32


def _make_spmm(d, chunk, nbuf, k, sb, stage_table=False):
    nc = EPT // chunk
    nt = nc // sb

    def body(*refs):
        (x_hbm, src_hbm, dst_hbm, zeros_hbm, out_hbm) = refs[:5]
        bufs = refs[5:5 + nbuf]
        src_v = refs[5 + nbuf:7 + nbuf]
        dst_v = refs[7 + nbuf:9 + nbuf]
        acc = refs[9 + nbuf]
        gsems = refs[10 + nbuf:10 + 2 * nbuf]
        ssems = refs[10 + 2 * nbuf:10 + 3 * nbuf]
        isems = refs[10 + 3 * nbuf:12 + 3 * nbuf]
        table = refs[12 + 3 * nbuf] if stage_table else x_hbm

        cid = lax.axis_index("c")
        sid = lax.axis_index("s")
        wid = sid * 2 + cid
        base = wid * nc

        def ifire(t):
            pltpu.async_copy(src_hbm.at[pl.ds(base + t * sb, sb)],
                             src_v[t % 2], isems[0])
            pltpu.async_copy(dst_hbm.at[pl.ds(base + t * sb, sb)],
                             dst_v[t % 2], isems[1])

        def iwait(t):
            pltpu.make_async_copy(src_hbm.at[pl.ds(base + t * sb, sb)],
                                  src_v[t % 2], isems[0]).wait()
            pltpu.make_async_copy(dst_hbm.at[pl.ds(base + t * sb, sb)],
                                  dst_v[t % 2], isems[1]).wait()

        def gstart(c):
            b = c % nbuf
            pltpu.async_copy(table.at[src_v[(c // sb) % 2].at[c % sb]],
                             bufs[b], gsems[b])

        def gwait(c):
            b = c % nbuf
            pltpu.make_async_copy(
                table.at[src_v[(c // sb) % 2].at[c % sb]],
                bufs[b], gsems[b]).wait()

        def sstart(c):
            b = c % nbuf
            pltpu.async_copy(bufs[b],
                             acc.at[dst_v[(c // sb) % 2].at[c % sb]],
                             ssems[b], add=True)

        def swait(c):
            b = c % nbuf
            pltpu.make_async_copy(
                bufs[b], acc.at[dst_v[(c // sb) % 2].at[c % sb]],
                ssems[b]).wait()

        ifire(0)
        iwait(0)
        if nt > 1:
            ifire(1)
        if stage_table:
            pltpu.sync_copy(x_hbm.at[pl.ds(sid * ROWS_PT, ROWS_PT)],
                            table.at[pl.ds(sid * ROWS_PT, ROWS_PT)])
        else:
            for c in range(nbuf):
                gstart(c)
        for j in range(ROWS_PT // 128):
            pltpu.sync_copy(zeros_hbm,
                            acc.at[pl.ds(sid * ROWS_PT + j * 128, 128)])
        plsc.subcore_barrier()
        if stage_table:
            for c in range(nbuf):
                gstart(c)
        if nt > 1:
            iwait(1)

        for j in range(nc):
            gwait(j)
            sstart(j)
            i = j - k
            if 0 <= i and i + nbuf < nc:
                swait(i)
                gstart(i + nbuf)
            if j >= k and (j - k + 1) % sb == 0:
                t = (j - k + 1) // sb + 1
                if t < nt:
                    ifire(t)
                    iwait(t)
        for i in range(nc - nbuf, nc):
            swait(i)
        plsc.subcore_barrier()
        r0 = sid * ROWS_PT
        pltpu.sync_copy(acc.at[pl.ds(r0, ROWS_PT)],
                        out_hbm.at[cid].at[pl.ds(r0, ROWS_PT)])

    scratch = (
        [pltpu.VMEM((chunk, d), jnp.float32)] * nbuf
        + [pltpu.VMEM((sb, chunk), jnp.int32)] * 4
        + [pltpu.VMEM_SHARED((NP, d), jnp.float32)]
        + [pltpu.SemaphoreType.DMA] * (2 * nbuf + 2)
        + ([pltpu.VMEM_SHARED((NP, d), jnp.float32)] if stage_table else [])
    )
    return functools.partial(
        pl.kernel,
        out_type=jax.ShapeDtypeStruct((2, NP, d), jnp.float32),
        mesh=_mesh,
        compiler_params=_sc_params,
        scratch_types=scratch,
    )(body)


_spmm128 = _make_spmm(128, 32, 8, 4, 32)
_spmm64 = _make_spmm(64, 128, 4, 2, 16, stage_table=True)


ROWB = 256


def _norm(d0, d1):
    deg = (d0 + d1)[:, 0:1]
    return lax.rsqrt(jnp.where(deg > 0.0, deg, 1.0))


def _tc_scale_body(f_ref, d0_ref, d1_ref, o_ref):
    o_ref[...] = f_ref[...] * _norm(d0_ref[...], d1_ref[...])


def _tc_mid_body(a0_ref, a1_ref, di0_ref, di1_ref, do0_ref, do1_ref,
                 w1_ref, b1_ref, w2_ref, o_ref):
    ni = _norm(di0_ref[...], di1_ref[...])
    a = (a0_ref[...] + a1_ref[...]) * ni
    h = jnp.dot(a, w1_ref[...], preferred_element_type=jnp.float32)
    h = jnp.maximum(h + b1_ref[...], 0.0)
    no = _norm(do0_ref[...], do1_ref[...])
    y = jnp.dot(h * no, w2_ref[...], preferred_element_type=jnp.float32)
    row = pl.program_id(0) * ROWB + lax.broadcasted_iota(
        jnp.int32, (ROWB, 1), 0)
    o_ref[...] = jnp.where(row < N, y, 0.0)


def _tc_final_body(g0_ref, g1_ref, di0_ref, di1_ref, b2_ref, o_ref):
    ni = _norm(di0_ref[...], di1_ref[...])
    o_ref[...] = (g0_ref[...] + g1_ref[...]) * ni + b2_ref[...]


def _rows_spec(d):
    return pl.BlockSpec((ROWB, d), lambda i: (i, 0))


def _full_spec(shape):
    return pl.BlockSpec(shape, lambda i: tuple(0 for _ in shape))


def kernel(features, edge_index, W1, b1, W2, b2):
    f32 = jnp.float32
    src = edge_index[0]
    dst = edge_index[1]
    pad = jnp.full((EP - E,), PAD_IDX, dtype=jnp.int32)
    src_f = jnp.concatenate([src, pad])
    dst_f = jnp.concatenate([dst, pad])
    src_p = src_f.reshape(EP // CHUNK, CHUNK)
    dst_p = dst_f.reshape(EP // CHUNK, CHUNK)
    src_p32 = src_f.reshape(EP // 32, 32)
    dst_p32 = dst_f.reshape(EP // 32, 32)

    ones_rows = jnp.zeros((2 * CHUNK, DW), f32).at[:CHUNK, 0].set(1.0)
    degp = _deg_kernel(src_p, dst_p, ones_rows)
    dO0, dI0 = degp[0, 0], degp[0, 1]
    dO1, dI1 = degp[1, 0], degp[1, 1]

    feats_p = jnp.pad(features, ((0, NP - N), (0, 0)))
    grid = (NP // ROWB,)
    xs = pl.pallas_call(
        _tc_scale_body,
        grid=grid,
        in_specs=[_rows_spec(128), _rows_spec(DW), _rows_spec(DW)],
        out_specs=_rows_spec(128),
        out_shape=jax.ShapeDtypeStruct((NP, 128), f32),
    )(feats_p, dO0, dO1)

    zeros128 = jnp.zeros((128, 128), f32)
    agg1 = _spmm128(xs, src_p32, dst_p32, zeros128)

    y = pl.pallas_call(
        _tc_mid_body,
        grid=grid,
        in_specs=[_rows_spec(128), _rows_spec(128),
                  _rows_spec(DW), _rows_spec(DW),
                  _rows_spec(DW), _rows_spec(DW),
                  _full_spec((128, 256)), _full_spec((1, 256)),
                  _full_spec((256, 64))],
        out_specs=_rows_spec(64),
        out_shape=jax.ShapeDtypeStruct((NP, 64), f32),
    )(agg1[0], agg1[1], dI0, dI1, dO0, dO1,
      W1, b1.reshape(1, 256), W2)

    zeros64 = jnp.zeros((128, 64), f32)
    agg2 = _spmm64(y, src_p, dst_p, zeros64)

    out = pl.pallas_call(
        _tc_final_body,
        grid=grid,
        in_specs=[_rows_spec(64), _rows_spec(64),
                  _rows_spec(DW), _rows_spec(DW),
                  _full_spec((1, 64))],
        out_specs=_rows_spec(64),
        out_shape=jax.ShapeDtypeStruct((NP, 64), f32),
    )(agg2[0], agg2[1], dI0, dI1, b2.reshape(1, 64))

    return out[:N]


# speedup vs baseline: 8.3903x; 1.4100x over previous
import functools

import jax
import jax.numpy as jnp
from jax import lax
from jax.experimental import pallas as pl
from jax.experimental.pallas import tpu as pltpu
from jax.experimental.pallas import tpu_sc as plsc

N = 10000
NP = 10240
E = 320000
EP = 327680
CHUNK = 128
CPT = EP // (32 * CHUNK)
ROWS_PT = NP // 16
PAD_IDX = NP - 1
DW = 8

_mesh = plsc.VectorSubcoreMesh(core_axis_name="c", subcore_axis_name="s")


def _deg_body(src_hbm, dst_hbm, ones_hbm, out_hbm,
              src_v, dst_v, ones_v, acc_s, acc_d, m0, m1, m2, m3):
    cid = lax.axis_index("c")
    sid = lax.axis_index("s")
    wid = sid * 2 + cid
    pltpu.sync_copy(src_hbm.at[pl.ds(wid * CPT, CPT)], src_v)
    pltpu.sync_copy(dst_hbm.at[pl.ds(wid * CPT, CPT)], dst_v)
    pltpu.sync_copy(ones_hbm, ones_v)
    for j in range(ROWS_PT // CHUNK):
        pltpu.sync_copy(ones_hbm.at[pl.ds(CHUNK, CHUNK)],
                        acc_s.at[pl.ds(sid * ROWS_PT + j * CHUNK, CHUNK)])
        pltpu.sync_copy(ones_hbm.at[pl.ds(CHUNK, CHUNK)],
                        acc_d.at[pl.ds(sid * ROWS_PT + j * CHUNK, CHUNK)])
    plsc.subcore_barrier()

    sems = (m0, m1, m2, m3)
    ones_row = ones_v.at[pl.ds(0, CHUNK)]

    def sadd(idx_row, acc, sem):
        pltpu.async_copy(ones_row, acc.at[idx_row], sem, add=True)

    def sdrain(idx_row, acc, sem):
        pltpu.make_async_copy(ones_row, acc.at[idx_row], sem).wait()

    del sems
    sadd(src_v.at[0], acc_s, m0)
    sadd(dst_v.at[0], acc_d, m1)
    sadd(src_v.at[1], acc_s, m2)
    sadd(dst_v.at[1], acc_d, m3)

    def chunk(i, carry):
        c = 2 * i
        sdrain(src_v.at[c], acc_s, m0)
        sdrain(dst_v.at[c], acc_d, m1)
        sadd(src_v.at[c + 2], acc_s, m0)
        sadd(dst_v.at[c + 2], acc_d, m1)
        sdrain(src_v.at[c + 1], acc_s, m2)
        sdrain(dst_v.at[c + 1], acc_d, m3)
        sadd(src_v.at[c + 3], acc_s, m2)
        sadd(dst_v.at[c + 3], acc_d, m3)
        return carry

    lax.fori_loop(0, (CPT - 2) // 2, chunk, 0)
    t = CPT - 2
    sdrain(src_v.at[t], acc_s, m0)
    sdrain(dst_v.at[t], acc_d, m1)
    sdrain(src_v.at[t + 1], acc_s, m2)
    sdrain(dst_v.at[t + 1], acc_d, m3)
    plsc.subcore_barrier()
    r0 = sid * ROWS_PT
    pltpu.sync_copy(acc_s.at[pl.ds(r0, ROWS_PT)],
                    out_hbm.at[cid, 0].at[pl.ds(r0, ROWS_PT)])
    pltpu.sync_copy(acc_d.at[pl.ds(r0, ROWS_PT)],
                    out_hbm.at[cid, 1].at[pl.ds(r0, ROWS_PT)])


_sc_params = pltpu.CompilerParams(use_tc_tiling_on_sc=False)

_deg_kernel = functools.partial(
    pl.kernel,
    out_type=jax.ShapeDtypeStruct((2, 2, NP, DW), jnp.float32),
    mesh=_mesh,
    compiler_params=_sc_params,
    scratch_types=[
        pltpu.VMEM((CPT, CHUNK), jnp.int32),
        pltpu.VMEM((CPT, CHUNK), jnp.int32),
        pltpu.VMEM((2 * CHUNK, DW), jnp.float32),
        pltpu.VMEM_SHARED((NP, DW), jnp.float32),
        pltpu.VMEM_SHARED((NP, DW), jnp.float32),
        pltpu.SemaphoreType.DMA,
        pltpu.SemaphoreType.DMA,
        pltpu.SemaphoreType.DMA,
        pltpu.SemaphoreType.DMA,
    ],
)(_deg_body)


EPT = EP // 32


def _make_spmm(d, chunk, nbuf, k, sb, stage_table=False):
    nc = EPT // chunk
    nt = nc // sb

    def body(*refs):
        (x_hbm, src_hbm, dst_hbm, zeros_hbm, out_hbm) = refs[:5]
        bufs = refs[5:5 + nbuf]
        src_v = refs[5 + nbuf:7 + nbuf]
        dst_v = refs[7 + nbuf:9 + nbuf]
        acc = refs[9 + nbuf]
        gsems = refs[10 + nbuf:10 + 2 * nbuf]
        ssems = refs[10 + 2 * nbuf:10 + 3 * nbuf]
        isems = refs[10 + 3 * nbuf:12 + 3 * nbuf]
        table = refs[12 + 3 * nbuf] if stage_table else x_hbm

        cid = lax.axis_index("c")
        sid = lax.axis_index("s")
        wid = sid * 2 + cid
        base = wid * nc

        def ifire(t):
            pltpu.async_copy(src_hbm.at[pl.ds(base + t * sb, sb)],
                             src_v[t % 2], isems[0])
            pltpu.async_copy(dst_hbm.at[pl.ds(base + t * sb, sb)],
                             dst_v[t % 2], isems[1])

        def iwait(t):
            pltpu.make_async_copy(src_hbm.at[pl.ds(base + t * sb, sb)],
                                  src_v[t % 2], isems[0]).wait()
            pltpu.make_async_copy(dst_hbm.at[pl.ds(base + t * sb, sb)],
                                  dst_v[t % 2], isems[1]).wait()

        def gstart(c):
            b = c % nbuf
            pltpu.async_copy(table.at[src_v[(c // sb) % 2].at[c % sb]],
                             bufs[b], gsems[b])

        def gwait(c):
            b = c % nbuf
            pltpu.make_async_copy(
                table.at[src_v[(c // sb) % 2].at[c % sb]],
                bufs[b], gsems[b]).wait()

        def sstart(c):
            b = c % nbuf
            pltpu.async_copy(bufs[b],
                             acc.at[dst_v[(c // sb) % 2].at[c % sb]],
                             ssems[b], add=True)

        def swait(c):
            b = c % nbuf
            pltpu.make_async_copy(
                bufs[b], acc.at[dst_v[(c // sb) % 2].at[c % sb]],
                ssems[b]).wait()

        ifire(0)
        iwait(0)
        if nt > 1:
            ifire(1)
        if stage_table:
            pltpu.sync_copy(x_hbm.at[pl.ds(sid * ROWS_PT, ROWS_PT)],
                            table.at[pl.ds(sid * ROWS_PT, ROWS_PT)])
        else:
            for c in range(nbuf):
                gstart(c)
        for j in range(ROWS_PT // 128):
            pltpu.sync_copy(zeros_hbm,
                            acc.at[pl.ds(sid * ROWS_PT + j * 128, 128)])
        plsc.subcore_barrier()
        if stage_table:
            for c in range(nbuf):
                gstart(c)
        if nt > 1:
            iwait(1)

        for j in range(nc):
            gwait(j)
            sstart(j)
            i = j - k
            if 0 <= i and i + nbuf < nc:
                swait(i)
                gstart(i + nbuf)
            if j >= k and (j - k + 1) % sb == 0:
                t = (j - k + 1) // sb + 1
                if t < nt:
                    ifire(t)
                    iwait(t)
        for i in range(nc - nbuf, nc):
            swait(i)
        plsc.subcore_barrier()
        r0 = sid * ROWS_PT
        pltpu.sync_copy(acc.at[pl.ds(r0, ROWS_PT)],
                        out_hbm.at[cid].at[pl.ds(r0, ROWS_PT)])

    scratch = (
        [pltpu.VMEM((chunk, d), jnp.float32)] * nbuf
        + [pltpu.VMEM((sb, chunk), jnp.int32)] * 4
        + [pltpu.VMEM_SHARED((NP, d), jnp.float32)]
        + [pltpu.SemaphoreType.DMA] * (2 * nbuf + 2)
        + ([pltpu.VMEM_SHARED((NP, d), jnp.float32)] if stage_table else [])
    )
    return functools.partial(
        pl.kernel,
        out_type=jax.ShapeDtypeStruct((2, NP, d), jnp.float32),
        mesh=_mesh,
        compiler_params=_sc_params,
        scratch_types=scratch,
    )(body)


_spmm64 = _make_spmm(64, 128, 4, 2, 16, stage_table=True)


def _make_spmm128_2pass(chunk=64, nbuf=8, k=4, sb=16):
    nc = EPT // chunk
    nt = nc // sb

    def body(*refs):
        (x0, x1, src_hbm, dst_hbm, zeros_hbm, out_hbm) = refs[:6]
        bufs = refs[6:6 + nbuf]
        src_v = refs[6 + nbuf:8 + nbuf]
        dst_v = refs[8 + nbuf:10 + nbuf]
        acc = refs[10 + nbuf]
        table = refs[11 + nbuf]
        gsems = refs[12 + nbuf:12 + 2 * nbuf]
        ssems = refs[12 + 2 * nbuf:12 + 3 * nbuf]
        isems = refs[12 + 3 * nbuf:14 + 3 * nbuf]

        cid = lax.axis_index("c")
        sid = lax.axis_index("s")
        wid = sid * 2 + cid
        base = wid * nc
        r0 = sid * ROWS_PT

        def ifire(t):
            pltpu.async_copy(src_hbm.at[pl.ds(base + t * sb, sb)],
                             src_v[t % 2], isems[0])
            pltpu.async_copy(dst_hbm.at[pl.ds(base + t * sb, sb)],
                             dst_v[t % 2], isems[1])

        def iwait(t):
            pltpu.make_async_copy(src_hbm.at[pl.ds(base + t * sb, sb)],
                                  src_v[t % 2], isems[0]).wait()
            pltpu.make_async_copy(dst_hbm.at[pl.ds(base + t * sb, sb)],
                                  dst_v[t % 2], isems[1]).wait()

        def gstart(c):
            b = c % nbuf
            pltpu.async_copy(table.at[src_v[(c // sb) % 2].at[c % sb]],
                             bufs[b], gsems[b])

        def gwait(c):
            b = c % nbuf
            pltpu.make_async_copy(
                table.at[src_v[(c // sb) % 2].at[c % sb]],
                bufs[b], gsems[b]).wait()

        def sstart(c):
            b = c % nbuf
            pltpu.async_copy(bufs[b],
                             acc.at[dst_v[(c // sb) % 2].at[c % sb]],
                             ssems[b], add=True)

        def swait(c):
            b = c % nbuf
            pltpu.make_async_copy(
                bufs[b], acc.at[dst_v[(c // sb) % 2].at[c % sb]],
                ssems[b]).wait()

        for p in range(2):
            xp = (x0, x1)[p]
            ifire(0)
            iwait(0)
            if nt > 1:
                ifire(1)
            pltpu.sync_copy(xp.at[pl.ds(r0, ROWS_PT)],
                            table.at[pl.ds(r0, ROWS_PT)])
            for j in range(ROWS_PT // 128):
                pltpu.sync_copy(zeros_hbm,
                                acc.at[pl.ds(r0 + j * 128, 128)])
            plsc.subcore_barrier()
            for c in range(nbuf):
                gstart(c)
            if nt > 1:
                iwait(1)
            for j in range(nc):
                gwait(j)
                sstart(j)
                i = j - k
                if 0 <= i and i + nbuf < nc:
                    swait(i)
                    gstart(i + nbuf)
                if j >= k and (j - k + 1) % sb == 0:
                    t = (j - k + 1) // sb + 1
                    if t < nt:
                        ifire(t)
                        iwait(t)
            for i in range(nc - nbuf, nc):
                swait(i)
            plsc.subcore_barrier()
            pltpu.sync_copy(acc.at[pl.ds(r0, ROWS_PT)],
                            out_hbm.at[cid, p].at[pl.ds(r0, ROWS_PT)])

    scratch = (
        [pltpu.VMEM((chunk, 64), jnp.float32)] * nbuf
        + [pltpu.VMEM((sb, chunk), jnp.int32)] * 4
        + [pltpu.VMEM_SHARED((NP, 64), jnp.float32)] * 2
        + [pltpu.SemaphoreType.DMA] * (2 * nbuf + 2)
    )
    return functools.partial(
        pl.kernel,
        out_type=jax.ShapeDtypeStruct((2, 2, NP, 64), jnp.float32),
        mesh=_mesh,
        compiler_params=_sc_params,
        scratch_types=scratch,
    )(body)


_spmm128 = _make_spmm128_2pass()


ROWB = 256


def _norm(d0, d1):
    deg = (d0 + d1)[:, 0:1]
    return lax.rsqrt(jnp.where(deg > 0.0, deg, 1.0))


def _tc_scale_body(f_ref, d0_ref, d1_ref, o0_ref, o1_ref):
    xs = f_ref[...] * _norm(d0_ref[...], d1_ref[...])
    o0_ref[...] = xs[:, :64]
    o1_ref[...] = xs[:, 64:]


def _tc_mid_body(a00_ref, a01_ref, a10_ref, a11_ref,
                 di0_ref, di1_ref, do0_ref, do1_ref,
                 w1_ref, b1_ref, w2_ref, o_ref):
    ni = _norm(di0_ref[...], di1_ref[...])
    a = jnp.concatenate([a00_ref[...] + a10_ref[...],
                         a01_ref[...] + a11_ref[...]], axis=1) * ni
    h = jnp.dot(a, w1_ref[...], preferred_element_type=jnp.float32)
    h = jnp.maximum(h + b1_ref[...], 0.0)
    no = _norm(do0_ref[...], do1_ref[...])
    y = jnp.dot(h * no, w2_ref[...], preferred_element_type=jnp.float32)
    row = pl.program_id(0) * ROWB + lax.broadcasted_iota(
        jnp.int32, (ROWB, 1), 0)
    o_ref[...] = jnp.where(row < N, y, 0.0)


def _tc_final_body(g0_ref, g1_ref, di0_ref, di1_ref, b2_ref, o_ref):
    ni = _norm(di0_ref[...], di1_ref[...])
    o_ref[...] = (g0_ref[...] + g1_ref[...]) * ni + b2_ref[...]


def _rows_spec(d):
    return pl.BlockSpec((ROWB, d), lambda i: (i, 0))


def _full_spec(shape):
    return pl.BlockSpec(shape, lambda i: tuple(0 for _ in shape))


def kernel(features, edge_index, W1, b1, W2, b2):
    f32 = jnp.float32
    src = edge_index[0]
    dst = edge_index[1]
    pad = jnp.full((EP - E,), PAD_IDX, dtype=jnp.int32)
    src_f = jnp.concatenate([src, pad])
    dst_f = jnp.concatenate([dst, pad])
    src_p = src_f.reshape(EP // CHUNK, CHUNK)
    dst_p = dst_f.reshape(EP // CHUNK, CHUNK)
    src_p64 = src_f.reshape(EP // 64, 64)
    dst_p64 = dst_f.reshape(EP // 64, 64)

    ones_rows = jnp.zeros((2 * CHUNK, DW), f32).at[:CHUNK, 0].set(1.0)
    degp = _deg_kernel(src_p, dst_p, ones_rows)
    dO0, dI0 = degp[0, 0], degp[0, 1]
    dO1, dI1 = degp[1, 0], degp[1, 1]

    feats_p = jnp.pad(features, ((0, NP - N), (0, 0)))
    grid = (NP // ROWB,)
    xs0, xs1 = pl.pallas_call(
        _tc_scale_body,
        grid=grid,
        in_specs=[_rows_spec(128), _rows_spec(DW), _rows_spec(DW)],
        out_specs=[_rows_spec(64), _rows_spec(64)],
        out_shape=[jax.ShapeDtypeStruct((NP, 64), f32),
                   jax.ShapeDtypeStruct((NP, 64), f32)],
    )(feats_p, dO0, dO1)

    zeros64 = jnp.zeros((128, 64), f32)
    agg1 = _spmm128(xs0, xs1, src_p64, dst_p64, zeros64)

    y = pl.pallas_call(
        _tc_mid_body,
        grid=grid,
        in_specs=[_rows_spec(64), _rows_spec(64),
                  _rows_spec(64), _rows_spec(64),
                  _rows_spec(DW), _rows_spec(DW),
                  _rows_spec(DW), _rows_spec(DW),
                  _full_spec((128, 256)), _full_spec((1, 256)),
                  _full_spec((256, 64))],
        out_specs=_rows_spec(64),
        out_shape=jax.ShapeDtypeStruct((NP, 64), f32),
    )(agg1[0, 0], agg1[0, 1], agg1[1, 0], agg1[1, 1],
      dI0, dI1, dO0, dO1, W1, b1.reshape(1, 256), W2)

    agg2 = _spmm64(y, src_p, dst_p, zeros64)

    out = pl.pallas_call(
        _tc_final_body,
        grid=grid,
        in_specs=[_rows_spec(64), _rows_spec(64),
                  _rows_spec(DW), _rows_spec(DW),
                  _full_spec((1, 64))],
        out_specs=_rows_spec(64),
        out_shape=jax.ShapeDtypeStruct((NP, 64), f32),
    )(agg2[0], agg2[1], dI0, dI1, b2.reshape(1, 64))

    return out[:N]
